# baseline (speedup 1.0000x reference)
import jax
import jax.numpy as jnp
from jax.experimental import pallas as pl
from jax.experimental.pallas import tpu as pltpu

VOCAB = 1000
BATCH = 1024
HIST = 50
ROWS = BATCH * HIST
BLOCK_R = 3200


def _onehot_block(ids_ref, out_ref):
    ids = ids_ref[0, 0, :]
    iota = jax.lax.broadcasted_iota(jnp.int32, (BLOCK_R, VOCAB), 1)
    out_ref[:, :] = (iota == ids[:, None]).astype(jnp.int8)


def kernel(input):
    ids = input.astype(jnp.int32).reshape(ROWS // BLOCK_R, 1, BLOCK_R)
    oh8 = pl.pallas_call(
        _onehot_block,
        grid=(ROWS // BLOCK_R,),
        in_specs=[pl.BlockSpec((1, 1, BLOCK_R), lambda i: (i, 0, 0))],
        out_specs=pl.BlockSpec((BLOCK_R, VOCAB), lambda i: (i, 0)),
        out_shape=jax.ShapeDtypeStruct((ROWS, VOCAB), jnp.int8),
    )(ids)
    return oh8.astype(jnp.float32).reshape(BATCH, HIST, VOCAB)
